# baseline (speedup 1.0000x reference)
import jax
import jax.numpy as jnp
from jax.experimental import pallas as pl

_BJ = 256


def _gcn_block(x_ref, adj_ref, w_ref, out_ref):
    j = pl.program_id(0)
    t = jax.lax.dot_general(
        x_ref[...], adj_ref[...],
        dimension_numbers=(((1,), (1,)), ((), ())),
        preferred_element_type=jnp.float32,
        precision=jax.lax.Precision.DEFAULT,
    )
    partial = jax.lax.dot_general(
        t, w_ref[...],
        dimension_numbers=(((1,), (0,)), ((), ())),
        preferred_element_type=jnp.float32,
        precision=jax.lax.Precision.DEFAULT,
    )

    @pl.when(j == 0)
    def _():
        out_ref[...] = partial

    @pl.when(j != 0)
    def _():
        out_ref[...] += partial


def kernel(x, adj, weight):
    batch, in_f = x.shape
    out_f = weight.shape[1]
    return pl.pallas_call(
        _gcn_block,
        grid=(in_f // _BJ,),
        in_specs=[
            pl.BlockSpec((batch, in_f), lambda j: (0, 0)),
            pl.BlockSpec((_BJ, in_f), lambda j: (j, 0)),
            pl.BlockSpec((_BJ, out_f), lambda j: (j, 0)),
        ],
        out_specs=pl.BlockSpec((batch, out_f), lambda j: (0, 0)),
        out_shape=jax.ShapeDtypeStruct((batch, out_f), jnp.float32),
    )(x, adj, weight)
